# baseline (speedup 1.0000x reference)
import dataclasses

import jax
import jax.numpy as jnp
from jax import lax
from jax.experimental import pallas as pl
from jax.experimental.pallas import tpu as pltpu
from jax.experimental.pallas import tpu_sc as plsc

_N = 10000
_P = 10
_E = 320000
_D = 128

_ROWS = 10240
_NW = 32
_CHUNK = 128
_CPW = 80
_EPAD = _NW * _CPW * _CHUNK
_RPT = _ROWS // 16


def _elu(v):
    return jnp.where(v > 0, v, jnp.exp(jnp.minimum(v, 0.0)) - 1.0)


def _tc1_body(x_ref, w_ref, b_ref, g_ref, bb_ref, vbh_ref, o_ref):
    x = x_ref[...]
    y = jnp.dot(x, w_ref[...], preferred_element_type=jnp.float32) + b_ref[...]
    mu = jnp.mean(y, axis=-1, keepdims=True)
    var = jnp.mean((y - mu) ** 2, axis=-1, keepdims=True)
    y = (y - mu) / jnp.sqrt(var + 1e-5) * g_ref[...] + bb_ref[...]
    o_ref[...] = _elu(y) + vbh_ref[...]


def _sc_body(h_hbm, src_hbm, dst_hbm, agg_hbm, deg_hbm,
             src_v, dst_v, buf, deg_v, agg_sh, sem):
    c = lax.axis_index("c")
    s = lax.axis_index("s")
    wid = (1 - c) * 16 + s
    zero16 = jnp.zeros((16,), jnp.float32)
    one16 = jnp.full((16,), 1.0, jnp.float32)

    @pl.loop(0, _CHUNK)
    def _(r):
        @pl.loop(0, _D // 16)
        def _(k):
            buf[r, pl.ds(k * 16, 16)] = zero16

    @pl.loop(0, _ROWS // 16)
    def _(k):
        deg_v[pl.ds(k * 16, 16)] = zero16

    @pl.loop(0, _RPT // _CHUNK)
    def _(i):
        pltpu.sync_copy(buf, agg_sh.at[pl.ds(s * _RPT + i * _CHUNK, _CHUNK)])

    pltpu.sync_copy(src_hbm.at[wid], src_v)
    pltpu.sync_copy(dst_hbm.at[wid], dst_v)
    plsc.subcore_barrier()

    @pl.loop(0, _CPW)
    def _(j):
        pltpu.async_copy(h_hbm.at[src_v.at[j]], buf, sem).wait()
        pltpu.sync_copy(buf, agg_sh.at[dst_v.at[j]], add=True)

        @pl.loop(0, _CHUNK // 16)
        def _(k):
            idx16 = dst_v[j, pl.ds(k * 16, 16)]
            plsc.addupdate_scatter(deg_v, [idx16], one16)

    plsc.subcore_barrier()
    pltpu.sync_copy(agg_sh.at[pl.ds(s * _RPT, _RPT)],
                    agg_hbm.at[c, pl.ds(s * _RPT, _RPT)])
    pltpu.sync_copy(deg_v, deg_hbm.at[wid])


def _tc2_body(h_ref, agg_ref, deg_ref, map_ref, wes_ref, wen_ref,
              wa1_ref, wa2_ref, ba_ref, g_ref, b_ref, wo_ref, bo_ref,
              vbd_ref, o_ref):
    h = h_ref[...]
    agg = agg_ref[0] + agg_ref[1]
    ones32 = jnp.ones((_NW, 1), jnp.float32)
    deg = lax.dot_general(deg_ref[...], ones32, (((0,), (0,)), ((), ())),
                          preferred_element_type=jnp.float32)
    m = agg / jnp.maximum(deg, 1.0)
    h2 = (jnp.dot(h, wes_ref[...], preferred_element_type=jnp.float32)
          + jnp.dot(m, wen_ref[...], preferred_element_type=jnp.float32))
    h2 = _elu(h2) + vbd_ref[...]
    vx = h2[_N:_N + 16]
    vtab = jnp.dot(vx, wa2_ref[...], preferred_element_type=jnp.float32)
    onehot = (map_ref[...] ==
              lax.broadcasted_iota(jnp.int32, (_ROWS, 16), 1)).astype(jnp.float32)
    sel = jnp.dot(onehot, vtab, preferred_element_type=jnp.float32)
    y = jnp.dot(h2, wa1_ref[...], preferred_element_type=jnp.float32) + sel + ba_ref[...]
    mu = jnp.mean(y, axis=-1, keepdims=True)
    var = jnp.mean((y - mu) ** 2, axis=-1, keepdims=True)
    y = (y - mu) / jnp.sqrt(var + 1e-5) * g_ref[...] + b_ref[...]
    nx = _elu(y)
    o_ref[...] = jnp.dot(nx, wo_ref[...], preferred_element_type=jnp.float32) + bo_ref[...]


def _seg_sum(h, src, dst):
    mesh = plsc.VectorSubcoreMesh(core_axis_name="c", subcore_axis_name="s")
    cp = pltpu.CompilerParams()
    if "needs_layout_passes" in pltpu.CompilerParams.__dataclass_fields__:
        cp = dataclasses.replace(cp, needs_layout_passes=False)
    return pl.kernel(
        _sc_body,
        out_type=[jax.ShapeDtypeStruct((2, _ROWS, _D), jnp.float32),
                  jax.ShapeDtypeStruct((_NW, _ROWS), jnp.float32)],
        mesh=mesh,
        scratch_types=[
            pltpu.VMEM((_CPW, _CHUNK), jnp.int32),
            pltpu.VMEM((_CPW, _CHUNK), jnp.int32),
            pltpu.VMEM((_CHUNK, _D), jnp.float32),
            pltpu.VMEM((_ROWS,), jnp.float32),
            pltpu.VMEM_SHARED((_ROWS, _D), jnp.float32),
            pltpu.SemaphoreType.DMA,
        ],
        compiler_params=cp,
    )(h, src, dst)


def kernel(x, edge_index, mapping, W_in, b_in, ln_hid_g, ln_hid_b,
           W_enc_self, W_enc_nbr, ln_enc_g, ln_enc_b,
           W_aggr, b_aggr, W_out, b_out, vnode_embed, vb_hid, vb_dcd):
    f32 = jnp.float32
    pad = _ROWS - (_N + _P)
    x_eff = jnp.concatenate([x[:_N], vnode_embed, jnp.zeros((pad, _D), f32)], axis=0)
    vbh = jnp.zeros((_ROWS, _D), f32).at[_N:_N + _P].set(vb_hid)
    vbd = jnp.zeros((_ROWS, _D), f32).at[_N:_N + _P].set(vb_dcd)
    npad = _EPAD - _E
    src = jnp.concatenate(
        [edge_index[0], jnp.zeros((npad,), jnp.int32)]).reshape(_NW, _CPW, _CHUNK)
    pad_dst = _N + _P + (jnp.arange(npad, dtype=jnp.int32) % (_ROWS - (_N + _P)))
    dst = jnp.concatenate(
        [edge_index[1], pad_dst]).reshape(_NW, _CPW, _CHUNK)
    map_pad = jnp.concatenate(
        [mapping, jnp.zeros((_ROWS - _N,), jnp.int32)]).reshape(_ROWS, 1)

    h = pl.pallas_call(
        _tc1_body, out_shape=jax.ShapeDtypeStruct((_ROWS, _D), f32))(
        x_eff, W_in, b_in.reshape(1, _D), ln_hid_g.reshape(1, _D),
        ln_hid_b.reshape(1, _D), vbh)

    agg, deg = _seg_sum(h, src, dst)

    out = pl.pallas_call(
        _tc2_body, out_shape=jax.ShapeDtypeStruct((_ROWS, _D), f32))(
        h, agg, deg, map_pad, W_enc_self, W_enc_nbr,
        W_aggr[:_D], W_aggr[_D:], b_aggr.reshape(1, _D),
        ln_enc_g.reshape(1, _D), ln_enc_b.reshape(1, _D),
        W_out, b_out.reshape(1, _D), vbd)
    out_n = out[:_N]
    return (out_n, jnp.float32(0.0), out_n, mapping)


# speedup vs baseline: 2.3361x; 2.3361x over previous
import dataclasses

import jax
import jax.numpy as jnp
from jax import lax
from jax.experimental import pallas as pl
from jax.experimental.pallas import tpu as pltpu
from jax.experimental.pallas import tpu_sc as plsc

_N = 10000
_P = 10
_E = 320000
_D = 128

_ROWS = 10240
_NW = 32
_CHUNK = 128
_CPW = 80
_EPAD = _NW * _CPW * _CHUNK
_RPT = _ROWS // 16


def _elu(v):
    return jnp.where(v > 0, v, jnp.exp(jnp.minimum(v, 0.0)) - 1.0)


def _tc1_body(x_ref, w_ref, b_ref, g_ref, bb_ref, vbh_ref, o_ref):
    x = x_ref[...]
    y = jnp.dot(x, w_ref[...], preferred_element_type=jnp.float32) + b_ref[...]
    mu = jnp.mean(y, axis=-1, keepdims=True)
    var = jnp.mean((y - mu) ** 2, axis=-1, keepdims=True)
    y = (y - mu) / jnp.sqrt(var + 1e-5) * g_ref[...] + bb_ref[...]
    o_ref[...] = _elu(y) + vbh_ref[...]


def _sc_body(h_hbm, src_hbm, dst_hbm, agg_hbm, deg_hbm,
             src_v, dst_v, buf, deg_v, agg_sh, sem):
    c = lax.axis_index("c")
    s = lax.axis_index("s")
    wid = c * 16 + s
    zero16 = jnp.zeros((16,), jnp.float32)
    one16 = jnp.full((16,), 1.0, jnp.float32)

    @pl.loop(0, _CHUNK)
    def _(r):
        @pl.loop(0, _D // 16)
        def _(k):
            buf[r, pl.ds(k * 16, 16)] = zero16

    @pl.loop(0, _ROWS // 16)
    def _(k):
        deg_v[pl.ds(k * 16, 16)] = zero16

    @pl.loop(0, _RPT // _CHUNK)
    def _(i):
        pltpu.sync_copy(buf, agg_sh.at[pl.ds(s * _RPT + i * _CHUNK, _CHUNK)])

    pltpu.sync_copy(src_hbm.at[wid], src_v)
    pltpu.sync_copy(dst_hbm.at[wid], dst_v)
    plsc.subcore_barrier()

    @pl.loop(0, _CPW)
    def _(j):
        pltpu.async_copy(h_hbm.at[src_v.at[j]], buf, sem).wait()
        pltpu.sync_copy(buf, agg_sh.at[dst_v.at[j]], add=True)

        @pl.loop(0, _CHUNK // 16)
        def _(k):
            idx16 = dst_v[j, pl.ds(k * 16, 16)]
            plsc.addupdate_scatter(deg_v, [idx16], one16)

    plsc.subcore_barrier()
    pltpu.sync_copy(agg_sh.at[pl.ds(s * _RPT, _RPT)],
                    agg_hbm.at[c, pl.ds(s * _RPT, _RPT)])
    pltpu.sync_copy(deg_v, deg_hbm.at[wid])


def _tc2_body(h_ref, agg_ref, deg_ref, map_ref, wes_ref, wen_ref,
              wa1_ref, wa2_ref, ba_ref, g_ref, b_ref, wo_ref, bo_ref,
              vbd_ref, o_ref):
    h = h_ref[...]
    agg = agg_ref[0] + agg_ref[1]
    ones32 = jnp.ones((_NW, 1), jnp.float32)
    deg = lax.dot_general(deg_ref[...], ones32, (((0,), (0,)), ((), ())),
                          preferred_element_type=jnp.float32)
    m = agg / jnp.maximum(deg, 1.0)
    h2 = (jnp.dot(h, wes_ref[...], preferred_element_type=jnp.float32)
          + jnp.dot(m, wen_ref[...], preferred_element_type=jnp.float32))
    h2 = _elu(h2) + vbd_ref[...]
    vx = h2[_N:_N + 16]
    vtab = jnp.dot(vx, wa2_ref[...], preferred_element_type=jnp.float32)
    onehot = (map_ref[...] ==
              lax.broadcasted_iota(jnp.int32, (_ROWS, 16), 1)).astype(jnp.float32)
    sel = jnp.dot(onehot, vtab, preferred_element_type=jnp.float32)
    y = jnp.dot(h2, wa1_ref[...], preferred_element_type=jnp.float32) + sel + ba_ref[...]
    mu = jnp.mean(y, axis=-1, keepdims=True)
    var = jnp.mean((y - mu) ** 2, axis=-1, keepdims=True)
    y = (y - mu) / jnp.sqrt(var + 1e-5) * g_ref[...] + b_ref[...]
    nx = _elu(y)
    o_ref[...] = jnp.dot(nx, wo_ref[...], preferred_element_type=jnp.float32) + bo_ref[...]


def _seg_sum(h, src, dst):
    mesh = plsc.VectorSubcoreMesh(core_axis_name="c", subcore_axis_name="s")
    cp = pltpu.CompilerParams()
    if "needs_layout_passes" in pltpu.CompilerParams.__dataclass_fields__:
        cp = dataclasses.replace(cp, needs_layout_passes=False)
    return pl.kernel(
        _sc_body,
        out_type=[jax.ShapeDtypeStruct((2, _ROWS, _D), jnp.float32),
                  jax.ShapeDtypeStruct((_NW, _ROWS), jnp.float32)],
        mesh=mesh,
        scratch_types=[
            pltpu.VMEM((_CPW, _CHUNK), jnp.int32),
            pltpu.VMEM((_CPW, _CHUNK), jnp.int32),
            pltpu.VMEM((_CHUNK, _D), jnp.float32),
            pltpu.VMEM((_ROWS,), jnp.float32),
            pltpu.VMEM_SHARED((_ROWS, _D), jnp.float32),
            pltpu.SemaphoreType.DMA,
        ],
        compiler_params=cp,
    )(h, src, dst)


def kernel(x, edge_index, mapping, W_in, b_in, ln_hid_g, ln_hid_b,
           W_enc_self, W_enc_nbr, ln_enc_g, ln_enc_b,
           W_aggr, b_aggr, W_out, b_out, vnode_embed, vb_hid, vb_dcd):
    f32 = jnp.float32
    pad = _ROWS - (_N + _P)
    x_eff = jnp.concatenate([x[:_N], vnode_embed, jnp.zeros((pad, _D), f32)], axis=0)
    vbh = jnp.zeros((_ROWS, _D), f32).at[_N:_N + _P].set(vb_hid)
    vbd = jnp.zeros((_ROWS, _D), f32).at[_N:_N + _P].set(vb_dcd)
    npad = _EPAD - _E
    pad_iota = jnp.arange(npad, dtype=jnp.int32)
    pad_src = pad_iota % _N
    pad_dst = _N + _P + (pad_iota % (_ROWS - (_N + _P)))
    src = jnp.concatenate(
        [edge_index[0], pad_src]).reshape(_NW, _CPW, _CHUNK)
    dst = jnp.concatenate(
        [edge_index[1], pad_dst]).reshape(_NW, _CPW, _CHUNK)
    map_pad = jnp.concatenate(
        [mapping, jnp.zeros((_ROWS - _N,), jnp.int32)]).reshape(_ROWS, 1)

    h = pl.pallas_call(
        _tc1_body, out_shape=jax.ShapeDtypeStruct((_ROWS, _D), f32))(
        x_eff, W_in, b_in.reshape(1, _D), ln_hid_g.reshape(1, _D),
        ln_hid_b.reshape(1, _D), vbh)

    agg, deg = _seg_sum(h, src, dst)

    out = pl.pallas_call(
        _tc2_body, out_shape=jax.ShapeDtypeStruct((_ROWS, _D), f32))(
        h, agg, deg, map_pad, W_enc_self, W_enc_nbr,
        W_aggr[:_D], W_aggr[_D:], b_aggr.reshape(1, _D),
        ln_enc_g.reshape(1, _D), ln_enc_b.reshape(1, _D),
        W_out, b_out.reshape(1, _D), vbd)
    out_n = out[:_N]
    return (out_n, jnp.float32(0.0), out_n, mapping)


# speedup vs baseline: 2.9852x; 1.2778x over previous
import dataclasses

import jax
import jax.numpy as jnp
from jax import lax
from jax.experimental import pallas as pl
from jax.experimental.pallas import tpu as pltpu
from jax.experimental.pallas import tpu_sc as plsc

_N = 10000
_P = 10
_E = 320000
_D = 128

_ROWS = 10240
_NW = 32
_CHUNK = 128
_CPW = 80
_BLK = 16
_EPAD = _NW * _CPW * _CHUNK
_RPT = _ROWS // 16


def _elu(v):
    return jnp.where(v > 0, v, jnp.exp(jnp.minimum(v, 0.0)) - 1.0)


def _tc1_body(x_ref, w_ref, b_ref, g_ref, bb_ref, vbh_ref, o_ref):
    x = x_ref[...]
    y = jnp.dot(x, w_ref[...], preferred_element_type=jnp.float32) + b_ref[...]
    mu = jnp.mean(y, axis=-1, keepdims=True)
    var = jnp.mean((y - mu) ** 2, axis=-1, keepdims=True)
    y = (y - mu) / jnp.sqrt(var + 1e-5) * g_ref[...] + bb_ref[...]
    o_ref[...] = _elu(y) + vbh_ref[...]


def _sc_body(h_hbm, src_hbm, dst_hbm, agg_hbm, deg_hbm,
             src_v, dst_v, buf, buf2, deg_v, agg_sh, sem, sem2):
    c = lax.axis_index("c")
    s = lax.axis_index("s")
    wid = c * 16 + s
    zero16 = jnp.zeros((16,), jnp.float32)
    one16 = jnp.full((16,), 1.0, jnp.float32)

    @pl.loop(0, _CHUNK)
    def _(r):
        @pl.loop(0, _D // 16)
        def _(k):
            buf[r, pl.ds(k * 16, 16)] = zero16

    @pl.loop(0, _ROWS // 16)
    def _(k):
        deg_v[pl.ds(k * 16, 16)] = zero16

    @pl.loop(0, _RPT // _CHUNK)
    def _(i):
        pltpu.sync_copy(buf, agg_sh.at[pl.ds(s * _RPT + i * _CHUNK, _CHUNK)])

    plsc.subcore_barrier()

    def _drain(j, b, sm):
        pltpu.make_async_copy(h_hbm.at[src_v.at[j]], b, sm).wait()

    def _consume(j, b):
        pltpu.sync_copy(b, agg_sh.at[dst_v.at[j]], add=True)

        @pl.loop(0, _CHUNK // 16)
        def _(k):
            idx16 = dst_v[j, pl.ds(k * 16, 16)]
            plsc.addupdate_scatter(deg_v, [idx16], one16)

    @pl.loop(0, _CPW // _BLK)
    def _(b):
        pltpu.sync_copy(src_hbm.at[wid, pl.ds(b * _BLK, _BLK)], src_v)
        pltpu.sync_copy(dst_hbm.at[wid, pl.ds(b * _BLK, _BLK)], dst_v)

        pltpu.async_copy(h_hbm.at[src_v.at[0]], buf, sem)

        @pl.loop(0, _BLK // 2)
        def _(jj):
            j0 = 2 * jj
            j1 = j0 + 1
            j2 = jnp.minimum(j0 + 2, _BLK - 1)
            pltpu.async_copy(h_hbm.at[src_v.at[j1]], buf2, sem2)
            _drain(j0, buf, sem)
            _consume(j0, buf)
            pltpu.async_copy(h_hbm.at[src_v.at[j2]], buf, sem)
            _drain(j1, buf2, sem2)
            _consume(j1, buf2)

        _drain(_BLK - 1, buf, sem)

    plsc.subcore_barrier()
    pltpu.sync_copy(agg_sh.at[pl.ds(s * _RPT, _RPT)],
                    agg_hbm.at[c, pl.ds(s * _RPT, _RPT)])
    pltpu.sync_copy(deg_v, deg_hbm.at[wid])


def _tc2_body(h_ref, agg_ref, deg_ref, map_ref, wes_ref, wen_ref,
              wa1_ref, wa2_ref, ba_ref, g_ref, b_ref, wo_ref, bo_ref,
              vbd_ref, o_ref):
    h = h_ref[...]
    agg = agg_ref[0] + agg_ref[1]
    ones32 = jnp.ones((_NW, 1), jnp.float32)
    deg = lax.dot_general(deg_ref[...], ones32, (((0,), (0,)), ((), ())),
                          preferred_element_type=jnp.float32)
    m = agg / jnp.maximum(deg, 1.0)
    h2 = (jnp.dot(h, wes_ref[...], preferred_element_type=jnp.float32)
          + jnp.dot(m, wen_ref[...], preferred_element_type=jnp.float32))
    h2 = _elu(h2) + vbd_ref[...]
    vx = h2[_N:_N + 16]
    vtab = jnp.dot(vx, wa2_ref[...], preferred_element_type=jnp.float32)
    onehot = (map_ref[...] ==
              lax.broadcasted_iota(jnp.int32, (_ROWS, 16), 1)).astype(jnp.float32)
    sel = jnp.dot(onehot, vtab, preferred_element_type=jnp.float32)
    y = jnp.dot(h2, wa1_ref[...], preferred_element_type=jnp.float32) + sel + ba_ref[...]
    mu = jnp.mean(y, axis=-1, keepdims=True)
    var = jnp.mean((y - mu) ** 2, axis=-1, keepdims=True)
    y = (y - mu) / jnp.sqrt(var + 1e-5) * g_ref[...] + b_ref[...]
    nx = _elu(y)
    o_ref[...] = jnp.dot(nx, wo_ref[...], preferred_element_type=jnp.float32) + bo_ref[...]


def _seg_sum(h, src, dst):
    mesh = plsc.VectorSubcoreMesh(core_axis_name="c", subcore_axis_name="s")
    cp = pltpu.CompilerParams()
    if "needs_layout_passes" in pltpu.CompilerParams.__dataclass_fields__:
        cp = dataclasses.replace(cp, needs_layout_passes=False)
    return pl.kernel(
        _sc_body,
        out_type=[jax.ShapeDtypeStruct((2, _ROWS, _D), jnp.float32),
                  jax.ShapeDtypeStruct((_NW, _ROWS), jnp.float32)],
        mesh=mesh,
        scratch_types=[
            pltpu.VMEM((_BLK, _CHUNK), jnp.int32),
            pltpu.VMEM((_BLK, _CHUNK), jnp.int32),
            pltpu.VMEM((_CHUNK, _D), jnp.float32),
            pltpu.VMEM((_CHUNK, _D), jnp.float32),
            pltpu.VMEM((_ROWS,), jnp.float32),
            pltpu.VMEM_SHARED((_ROWS, _D), jnp.float32),
            pltpu.SemaphoreType.DMA,
            pltpu.SemaphoreType.DMA,
        ],
        compiler_params=cp,
    )(h, src, dst)


def kernel(x, edge_index, mapping, W_in, b_in, ln_hid_g, ln_hid_b,
           W_enc_self, W_enc_nbr, ln_enc_g, ln_enc_b,
           W_aggr, b_aggr, W_out, b_out, vnode_embed, vb_hid, vb_dcd):
    f32 = jnp.float32
    pad = _ROWS - (_N + _P)
    x_eff = jnp.concatenate([x[:_N], vnode_embed, jnp.zeros((pad, _D), f32)], axis=0)
    vbh = jnp.zeros((_ROWS, _D), f32).at[_N:_N + _P].set(vb_hid)
    vbd = jnp.zeros((_ROWS, _D), f32).at[_N:_N + _P].set(vb_dcd)
    npad = _EPAD - _E
    pad_iota = jnp.arange(npad, dtype=jnp.int32)
    pad_src = pad_iota % _N
    pad_dst = _N + _P + (pad_iota % (_ROWS - (_N + _P)))
    src = jnp.concatenate(
        [edge_index[0], pad_src]).reshape(_NW, _CPW, _CHUNK)
    dst = jnp.concatenate(
        [edge_index[1], pad_dst]).reshape(_NW, _CPW, _CHUNK)
    map_pad = jnp.concatenate(
        [mapping, jnp.zeros((_ROWS - _N,), jnp.int32)]).reshape(_ROWS, 1)

    h = pl.pallas_call(
        _tc1_body, out_shape=jax.ShapeDtypeStruct((_ROWS, _D), f32))(
        x_eff, W_in, b_in.reshape(1, _D), ln_hid_g.reshape(1, _D),
        ln_hid_b.reshape(1, _D), vbh)

    agg, deg = _seg_sum(h, src, dst)

    out = pl.pallas_call(
        _tc2_body, out_shape=jax.ShapeDtypeStruct((_ROWS, _D), f32))(
        h, agg, deg, map_pad, W_enc_self, W_enc_nbr,
        W_aggr[:_D], W_aggr[_D:], b_aggr.reshape(1, _D),
        ln_enc_g.reshape(1, _D), ln_enc_b.reshape(1, _D),
        W_out, b_out.reshape(1, _D), vbd)
    out_n = out[:_N]
    return (out_n, jnp.float32(0.0), out_n, mapping)


# speedup vs baseline: 3.0044x; 1.0064x over previous
import dataclasses

import jax
import jax.numpy as jnp
from jax import lax
from jax.experimental import pallas as pl
from jax.experimental.pallas import tpu as pltpu
from jax.experimental.pallas import tpu_sc as plsc

_N = 10000
_P = 10
_E = 320000
_D = 128

_ROWS = 10240
_NW = 32
_CHUNK = 128
_CPW = 80
_BLK = 16
_EPAD = _NW * _CPW * _CHUNK
_RPT = _ROWS // 16


def _elu(v):
    return jnp.where(v > 0, v, jnp.exp(jnp.minimum(v, 0.0)) - 1.0)


def _tc1_body(x_ref, w_ref, b_ref, g_ref, bb_ref, vbh_ref, o_ref):
    x = x_ref[...]
    y = jnp.dot(x, w_ref[...], preferred_element_type=jnp.float32) + b_ref[...]
    mu = jnp.mean(y, axis=-1, keepdims=True)
    var = jnp.mean((y - mu) ** 2, axis=-1, keepdims=True)
    y = (y - mu) / jnp.sqrt(var + 1e-5) * g_ref[...] + bb_ref[...]
    o_ref[...] = _elu(y) + vbh_ref[...]


def _sc_body(h_hbm, src_hbm, dst_hbm, agg_hbm, deg_hbm,
             src_v, dst_v, buf, buf2, deg_v, agg_sh, sem, sem2):
    c = lax.axis_index("c")
    s = lax.axis_index("s")
    wid = c * 16 + s
    zero16 = jnp.zeros((16,), jnp.float32)
    one16 = jnp.full((16,), 1.0, jnp.float32)

    @pl.loop(0, _CHUNK)
    def _(r):
        @pl.loop(0, _D // 16)
        def _(k):
            buf[r, pl.ds(k * 16, 16)] = zero16

    @pl.loop(0, _ROWS // 16)
    def _(k):
        deg_v[pl.ds(k * 16, 16)] = zero16

    @pl.loop(0, _RPT // _CHUNK)
    def _(i):
        pltpu.sync_copy(buf, agg_sh.at[pl.ds(s * _RPT + i * _CHUNK, _CHUNK)])

    plsc.subcore_barrier()

    def _drain(j, b, sm):
        pltpu.make_async_copy(h_hbm.at[src_v.at[j]], b, sm).wait()

    def _consume(j, b):
        pltpu.sync_copy(b, agg_sh.at[dst_v.at[j]], add=True)

        @pl.loop(0, _CHUNK // 16)
        def _(k):
            idx16 = dst_v[j, pl.ds(k * 16, 16)]
            plsc.addupdate_scatter(deg_v, [idx16], one16)

    @pl.loop(0, _CPW // _BLK)
    def _(b):
        pltpu.sync_copy(src_hbm.at[wid, pl.ds(b * _BLK, _BLK)], src_v)
        pltpu.sync_copy(dst_hbm.at[wid, pl.ds(b * _BLK, _BLK)], dst_v)

        pltpu.async_copy(h_hbm.at[src_v.at[0]], buf, sem)

        @pl.loop(0, _BLK // 2)
        def _(jj):
            j0 = 2 * jj
            j1 = j0 + 1
            j2 = jnp.minimum(j0 + 2, _BLK - 1)
            pltpu.async_copy(h_hbm.at[src_v.at[j1]], buf2, sem2)
            _drain(j0, buf, sem)
            _consume(j0, buf)
            pltpu.async_copy(h_hbm.at[src_v.at[j2]], buf, sem)
            _drain(j1, buf2, sem2)
            _consume(j1, buf2)

        _drain(_BLK - 1, buf, sem)

    plsc.subcore_barrier()
    pltpu.sync_copy(agg_sh.at[pl.ds(s * _RPT, _RPT)],
                    agg_hbm.at[c, pl.ds(s * _RPT, _RPT)])
    pltpu.sync_copy(deg_v, deg_hbm.at[wid])


def _tc2_body(h_ref, agg_ref, deg_ref, map_ref, wes_ref, wen_ref,
              wa1_ref, wa2_ref, ba_ref, g_ref, b_ref, wo_ref, bo_ref,
              vbd_ref, o_ref):
    h = h_ref[...]
    agg = agg_ref[0] + agg_ref[1]
    ones32 = jnp.ones((_NW, 1), jnp.float32)
    deg = lax.dot_general(deg_ref[...], ones32, (((0,), (0,)), ((), ())),
                          preferred_element_type=jnp.float32)
    m = agg / jnp.maximum(deg, 1.0)
    h2 = (jnp.dot(h, wes_ref[...], preferred_element_type=jnp.float32)
          + jnp.dot(m, wen_ref[...], preferred_element_type=jnp.float32))
    h2 = _elu(h2) + vbd_ref[...]
    vx = h2[_N:_N + 16]
    vtab = jnp.dot(vx, wa2_ref[...], preferred_element_type=jnp.float32)
    onehot = (map_ref[...] ==
              lax.broadcasted_iota(jnp.int32, (_ROWS, 16), 1)).astype(jnp.float32)
    sel = jnp.dot(onehot, vtab, preferred_element_type=jnp.float32)
    y = jnp.dot(h2, wa1_ref[...], preferred_element_type=jnp.float32) + sel + ba_ref[...]
    mu = jnp.mean(y, axis=-1, keepdims=True)
    var = jnp.mean((y - mu) ** 2, axis=-1, keepdims=True)
    y = (y - mu) / jnp.sqrt(var + 1e-5) * g_ref[...] + b_ref[...]
    nx = _elu(y)
    o = jnp.dot(nx, wo_ref[...], preferred_element_type=jnp.float32) + bo_ref[...]
    o_ref[...] = o[:_N]


def _seg_sum(h, src, dst):
    mesh = plsc.VectorSubcoreMesh(core_axis_name="c", subcore_axis_name="s")
    cp = pltpu.CompilerParams()
    if "needs_layout_passes" in pltpu.CompilerParams.__dataclass_fields__:
        cp = dataclasses.replace(cp, needs_layout_passes=False)
    return pl.kernel(
        _sc_body,
        out_type=[jax.ShapeDtypeStruct((2, _ROWS, _D), jnp.float32),
                  jax.ShapeDtypeStruct((_NW, _ROWS), jnp.float32)],
        mesh=mesh,
        scratch_types=[
            pltpu.VMEM((_BLK, _CHUNK), jnp.int32),
            pltpu.VMEM((_BLK, _CHUNK), jnp.int32),
            pltpu.VMEM((_CHUNK, _D), jnp.float32),
            pltpu.VMEM((_CHUNK, _D), jnp.float32),
            pltpu.VMEM((_ROWS,), jnp.float32),
            pltpu.VMEM_SHARED((_ROWS, _D), jnp.float32),
            pltpu.SemaphoreType.DMA,
            pltpu.SemaphoreType.DMA,
        ],
        compiler_params=cp,
    )(h, src, dst)


def kernel(x, edge_index, mapping, W_in, b_in, ln_hid_g, ln_hid_b,
           W_enc_self, W_enc_nbr, ln_enc_g, ln_enc_b,
           W_aggr, b_aggr, W_out, b_out, vnode_embed, vb_hid, vb_dcd):
    f32 = jnp.float32
    pad = _ROWS - (_N + _P)
    x_eff = jnp.concatenate([x[:_N], vnode_embed, jnp.zeros((pad, _D), f32)], axis=0)
    vbh = jnp.zeros((_ROWS, _D), f32).at[_N:_N + _P].set(vb_hid)
    vbd = jnp.zeros((_ROWS, _D), f32).at[_N:_N + _P].set(vb_dcd)
    npad = _EPAD - _E
    pad_iota = jnp.arange(npad, dtype=jnp.int32)
    pad_src = pad_iota % _N
    pad_dst = _N + _P + (pad_iota % (_ROWS - (_N + _P)))
    src = jnp.concatenate(
        [edge_index[0], pad_src]).reshape(_NW, _CPW, _CHUNK)
    dst = jnp.concatenate(
        [edge_index[1], pad_dst]).reshape(_NW, _CPW, _CHUNK)
    map_pad = jnp.concatenate(
        [mapping, jnp.zeros((_ROWS - _N,), jnp.int32)]).reshape(_ROWS, 1)

    h = pl.pallas_call(
        _tc1_body, out_shape=jax.ShapeDtypeStruct((_ROWS, _D), f32))(
        x_eff, W_in, b_in.reshape(1, _D), ln_hid_g.reshape(1, _D),
        ln_hid_b.reshape(1, _D), vbh)

    agg, deg = _seg_sum(h, src, dst)

    out = pl.pallas_call(
        _tc2_body, out_shape=jax.ShapeDtypeStruct((_N, _D), f32))(
        h, agg, deg, map_pad, W_enc_self, W_enc_nbr,
        W_aggr[:_D], W_aggr[_D:], b_aggr.reshape(1, _D),
        ln_enc_g.reshape(1, _D), ln_enc_b.reshape(1, _D),
        W_out, b_out.reshape(1, _D), vbd)
    return (out, jnp.float32(0.0), out, mapping)


# speedup vs baseline: 3.3225x; 1.1059x over previous
import dataclasses

import jax
import jax.numpy as jnp
from jax import lax
from jax.experimental import pallas as pl
from jax.experimental.pallas import tpu as pltpu
from jax.experimental.pallas import tpu_sc as plsc

_N = 10000
_P = 10
_E = 320000
_D = 128

_ROWS = 10240
_NW = 32
_CHUNK = 128
_CPW = 80
_BLK = 16
_EPAD = _NW * _CPW * _CHUNK
_RPT = _ROWS // 16


def _elu(v):
    return jnp.where(v > 0, v, jnp.exp(jnp.minimum(v, 0.0)) - 1.0)


def _tc1_body(x_ref, vne_ref, w_ref, b_ref, g_ref, bb_ref, vbh_ref, o_ref):
    def enc(v):
        y = jnp.dot(v, w_ref[...], preferred_element_type=jnp.float32) + b_ref[...]
        mu = jnp.mean(y, axis=-1, keepdims=True)
        var = jnp.mean((y - mu) ** 2, axis=-1, keepdims=True)
        y = (y - mu) / jnp.sqrt(var + 1e-5) * g_ref[...] + bb_ref[...]
        return _elu(y)

    res = enc(x_ref[...])
    vres = enc(vne_ref[...]) + vbh_ref[...]
    o_ref[pl.ds(0, _N)] = res[:_N]
    o_ref[pl.ds(_N, _ROWS - _N)] = jnp.concatenate(
        [vres, jnp.zeros((_ROWS - _N - _P, _D), jnp.float32)], axis=0)


def _sc_body(h_hbm, src_hbm, dst_hbm, agg_hbm, deg_hbm,
             src_v, dst_v, buf, buf2, deg_v, agg_sh, sem, sem2):
    c = lax.axis_index("c")
    s = lax.axis_index("s")
    wid = c * 16 + s
    zero16 = jnp.zeros((16,), jnp.float32)
    one16 = jnp.full((16,), 1.0, jnp.float32)

    @pl.loop(0, _CHUNK)
    def _(r):
        @pl.loop(0, _D // 16)
        def _(k):
            buf[r, pl.ds(k * 16, 16)] = zero16

    @pl.loop(0, _ROWS // 16)
    def _(k):
        deg_v[pl.ds(k * 16, 16)] = zero16

    @pl.loop(0, _RPT // _CHUNK)
    def _(i):
        pltpu.sync_copy(buf, agg_sh.at[pl.ds(s * _RPT + i * _CHUNK, _CHUNK)])

    plsc.subcore_barrier()

    def _drain(j, b, sm):
        pltpu.make_async_copy(h_hbm.at[src_v.at[j]], b, sm).wait()

    def _consume(j, b):
        pltpu.sync_copy(b, agg_sh.at[dst_v.at[j]], add=True)

        @pl.loop(0, _CHUNK // 16)
        def _(k):
            idx16 = dst_v[j, pl.ds(k * 16, 16)]
            plsc.addupdate_scatter(deg_v, [idx16], one16)

    @pl.loop(0, _CPW // _BLK)
    def _(b):
        pltpu.sync_copy(src_hbm.at[wid, pl.ds(b * _BLK, _BLK)], src_v)
        pltpu.sync_copy(dst_hbm.at[wid, pl.ds(b * _BLK, _BLK)], dst_v)

        pltpu.async_copy(h_hbm.at[src_v.at[0]], buf, sem)

        @pl.loop(0, _BLK // 2)
        def _(jj):
            j0 = 2 * jj
            j1 = j0 + 1
            j2 = jnp.minimum(j0 + 2, _BLK - 1)
            pltpu.async_copy(h_hbm.at[src_v.at[j1]], buf2, sem2)
            _drain(j0, buf, sem)
            _consume(j0, buf)
            pltpu.async_copy(h_hbm.at[src_v.at[j2]], buf, sem)
            _drain(j1, buf2, sem2)
            _consume(j1, buf2)

        _drain(_BLK - 1, buf, sem)

    plsc.subcore_barrier()
    pltpu.sync_copy(agg_sh.at[pl.ds(s * _RPT, _RPT)],
                    agg_hbm.at[c, pl.ds(s * _RPT, _RPT)])
    pltpu.sync_copy(deg_v, deg_hbm.at[wid])


def _tc2_body(h_ref, agg_ref, deg_ref, map_ref, wes_ref, wen_ref,
              wa1_ref, wa2_ref, ba_ref, g_ref, b_ref, wo_ref, bo_ref,
              vbd_ref, o_ref):
    h = h_ref[...]
    agg = agg_ref[0] + agg_ref[1]
    ones32 = jnp.ones((_NW, 1), jnp.float32)
    deg = lax.dot_general(deg_ref[...], ones32, (((0,), (0,)), ((), ())),
                          preferred_element_type=jnp.float32)
    m = agg / jnp.maximum(deg, 1.0)
    h2 = (jnp.dot(h, wes_ref[...], preferred_element_type=jnp.float32)
          + jnp.dot(m, wen_ref[...], preferred_element_type=jnp.float32))
    vbd16 = jnp.concatenate(
        [vbd_ref[...], jnp.zeros((16 - _P, _D), jnp.float32)], axis=0)
    rowi = lax.broadcasted_iota(jnp.int32, (_ROWS, 16), 0)
    coli = lax.broadcasted_iota(jnp.int32, (_ROWS, 16), 1)
    ohv = (rowi - _N == coli).astype(jnp.float32)
    h2 = _elu(h2) + jnp.dot(ohv, vbd16, preferred_element_type=jnp.float32)
    vx = h2[_N:_N + 16]
    vtab = jnp.dot(vx, wa2_ref[...], preferred_element_type=jnp.float32)
    onehot = (map_ref[...] ==
              lax.broadcasted_iota(jnp.int32, (_ROWS, 16), 1)).astype(jnp.float32)
    sel = jnp.dot(onehot, vtab, preferred_element_type=jnp.float32)
    y = jnp.dot(h2, wa1_ref[...], preferred_element_type=jnp.float32) + sel + ba_ref[...]
    mu = jnp.mean(y, axis=-1, keepdims=True)
    var = jnp.mean((y - mu) ** 2, axis=-1, keepdims=True)
    y = (y - mu) / jnp.sqrt(var + 1e-5) * g_ref[...] + b_ref[...]
    nx = _elu(y)
    o = jnp.dot(nx, wo_ref[...], preferred_element_type=jnp.float32) + bo_ref[...]
    o_ref[...] = o[:_N]


def _seg_sum(h, src, dst):
    mesh = plsc.VectorSubcoreMesh(core_axis_name="c", subcore_axis_name="s")
    cp = pltpu.CompilerParams()
    if "needs_layout_passes" in pltpu.CompilerParams.__dataclass_fields__:
        cp = dataclasses.replace(cp, needs_layout_passes=False)
    return pl.kernel(
        _sc_body,
        out_type=[jax.ShapeDtypeStruct((2, _ROWS, _D), jnp.float32),
                  jax.ShapeDtypeStruct((_NW, _ROWS), jnp.float32)],
        mesh=mesh,
        scratch_types=[
            pltpu.VMEM((_BLK, _CHUNK), jnp.int32),
            pltpu.VMEM((_BLK, _CHUNK), jnp.int32),
            pltpu.VMEM((_CHUNK, _D), jnp.float32),
            pltpu.VMEM((_CHUNK, _D), jnp.float32),
            pltpu.VMEM((_ROWS,), jnp.float32),
            pltpu.VMEM_SHARED((_ROWS, _D), jnp.float32),
            pltpu.SemaphoreType.DMA,
            pltpu.SemaphoreType.DMA,
        ],
        compiler_params=cp,
    )(h, src, dst)


def kernel(x, edge_index, mapping, W_in, b_in, ln_hid_g, ln_hid_b,
           W_enc_self, W_enc_nbr, ln_enc_g, ln_enc_b,
           W_aggr, b_aggr, W_out, b_out, vnode_embed, vb_hid, vb_dcd):
    f32 = jnp.float32
    npad = _EPAD - _E
    pad_iota = jnp.arange(npad, dtype=jnp.int32)
    pad_src = pad_iota % _N
    pad_dst = _N + _P + (pad_iota % (_ROWS - (_N + _P)))
    src = jnp.concatenate(
        [edge_index[0], pad_src]).reshape(_NW, _CPW, _CHUNK)
    dst = jnp.concatenate(
        [edge_index[1], pad_dst]).reshape(_NW, _CPW, _CHUNK)
    map_pad = jnp.concatenate(
        [mapping, jnp.zeros((_ROWS - _N,), jnp.int32)]).reshape(_ROWS, 1)

    h = pl.pallas_call(
        _tc1_body, out_shape=jax.ShapeDtypeStruct((_ROWS, _D), f32))(
        x[:_N], vnode_embed, W_in, b_in.reshape(1, _D), ln_hid_g.reshape(1, _D),
        ln_hid_b.reshape(1, _D), vb_hid)

    agg, deg = _seg_sum(h, src, dst)

    out = pl.pallas_call(
        _tc2_body, out_shape=jax.ShapeDtypeStruct((_N, _D), f32))(
        h, agg, deg, map_pad, W_enc_self, W_enc_nbr,
        W_aggr[:_D], W_aggr[_D:], b_aggr.reshape(1, _D),
        ln_enc_g.reshape(1, _D), ln_enc_b.reshape(1, _D),
        W_out, b_out.reshape(1, _D), vb_dcd)
    return (out, jnp.float32(0.0), out, mapping)
